# baseline (speedup 1.0000x reference)
import functools
import math

import jax
import jax.numpy as jnp
from jax import lax
from jax.experimental import pallas as pl
from jax.experimental.pallas import tpu as pltpu
from jax.experimental.pallas import tpu_sc as plsc

NODE_DIM = 10
HIDDEN = 64
N_LAYERS = 4
K_NN = 16
TOP_FRACTION = 0.4
MASK_SHARPNESS = 5.0
EDGE_DIM = 4

_INV_SQRT2 = 0.7071067811865476


def _gelu_exact(x):
    return 0.5 * x * (1.0 + lax.erf(x * _INV_SQRT2))


def _knn_body(cols_hbm, out_hbm, px, py, pz, buf, *, M, rows_per_w, nc):
    c = lax.axis_index("c")
    s = lax.axis_index("s")
    wid = s * nc + c
    base = wid * rows_per_w
    b = base // M
    lbase = base - b * M
    cbase = b * 3 * M
    pltpu.sync_copy(cols_hbm.at[pl.ds(cbase, M)], px.at[pl.ds(0, M)])
    pltpu.sync_copy(cols_hbm.at[pl.ds(cbase + M, M)], py.at[pl.ds(0, M)])
    pltpu.sync_copy(cols_hbm.at[pl.ds(cbase + 2 * M, M)], pz.at[pl.ds(0, M)])

    nchunks = M // 16
    inf = jnp.float32(jnp.inf)
    lane = lax.iota(jnp.int32, 16)

    def row_body(r, _):
        il = lbase + r
        xi = px[pl.ds(il, 16)][0]
        yi = py[pl.ds(il, 16)][0]
        zi = pz[pl.ds(il, 16)][0]

        def chunk_body(ci, carry):
            bd, bi, bound = carry
            off = ci * 16
            xs = px[pl.ds(off, 16)]
            ys = py[pl.ds(off, 16)]
            zs = pz[pl.ds(off, 16)]
            dx = xs - xi
            dy = ys - yi
            dz = zs - zi
            d = dx * dx + dy * dy + dz * dz
            idxs = off + lane
            d = jnp.where(idxs == il, inf, d)

            def merge(args):
                bd0, bi0, d0, i0 = args
                cd, cidx = plsc.sort_key_val(d0, i0, descending=True)
                take = cd < bd0
                nd = jnp.where(take, cd, bd0)
                ni = jnp.where(take, cidx, bi0)
                nd, ni = plsc.sort_key_val(nd, ni)
                nb = jnp.broadcast_to(nd[15], (16,))
                return nd, ni, nb

            def keep(args):
                bd0, bi0, _, _ = args
                return bd0, bi0, bound

            cnt = plsc.all_reduce_population_count(d < bound)
            pred = (cnt > 0) if jnp.ndim(cnt) == 0 else (cnt[0] > 0)
            return lax.cond(pred, merge, keep, (bd, bi, d, idxs))

        init = (jnp.full((16,), inf, jnp.float32),
                jnp.zeros((16,), jnp.int32),
                jnp.full((16,), inf, jnp.float32))
        _, bi, _ = lax.fori_loop(0, nchunks, chunk_body, init)
        buf[pl.ds(r * 16, 16)] = bi
        return 0

    lax.fori_loop(0, rows_per_w, row_body, 0)
    pltpu.sync_copy(buf, out_hbm.at[pl.ds(base * K_NN, rows_per_w * K_NN)])


def _sc_knn(pos_sel, B, M):
    info = plsc.get_sparse_core_info()
    nc, ns = info.num_cores, info.num_subcores
    nw = nc * ns
    rows_per_w = (B * M) // nw
    cols = pos_sel.transpose(0, 2, 1).reshape(-1)
    mesh = plsc.VectorSubcoreMesh(core_axis_name="c", subcore_axis_name="s")
    knn = pl.kernel(
        functools.partial(_knn_body, M=M, rows_per_w=rows_per_w, nc=nc),
        out_type=jax.ShapeDtypeStruct((B * M * K_NN,), jnp.int32),
        mesh=mesh,
        compiler_params=pltpu.CompilerParams(needs_layout_passes=False),
        scratch_types=[
            pltpu.VMEM((M + 16,), jnp.float32),
            pltpu.VMEM((M + 16,), jnp.float32),
            pltpu.VMEM((M + 16,), jnp.float32),
            pltpu.VMEM((rows_per_w * K_NN,), jnp.int32),
        ],
    )(cols)
    return knn.reshape(B * M, K_NN)


def _edge_body(ag_ref, bv_ref, ea_ref, wcb_ref, b1_ref, o_ref):
    call = jnp.dot(ea_ref[...], wcb_ref[...],
                   preferred_element_type=jnp.float32)
    base = ag_ref[...] + call
    bvb = bv_ref[...] + b1_ref[...]
    for j in range(K_NN):
        sl = slice(j * HIDDEN, (j + 1) * HIDDEN)
        o_ref[:, sl] = _gelu_exact(base[:, sl] + bvb)


def _edge_stage(ag, bv, ea, wcb, b1):
    n = ag.shape[0]
    R = 800
    grid = n // R
    row = lambda i: (i, 0)
    fixed = lambda i: (0, 0)
    return pl.pallas_call(
        _edge_body,
        grid=(grid,),
        in_specs=[
            pl.BlockSpec((R, K_NN * HIDDEN), row),
            pl.BlockSpec((R, HIDDEN), row),
            pl.BlockSpec((R, K_NN * EDGE_DIM), row),
            pl.BlockSpec((K_NN * EDGE_DIM, K_NN * HIDDEN), fixed),
            pl.BlockSpec((1, HIDDEN), fixed),
        ],
        out_specs=pl.BlockSpec((R, K_NN * HIDDEN), row),
        out_shape=jax.ShapeDtypeStruct((n, K_NN * HIDDEN), jnp.float32),
    )(ag, bv, ea, wcb, b1)


def _node_body(s_ref, db2_ref, xres_ref, w2t_ref, g_ref, bb_ref, o_ref, *,
               with_res):
    out = jnp.dot(s_ref[...], w2t_ref[...],
                  preferred_element_type=jnp.float32) + db2_ref[...]
    mu = jnp.mean(out, axis=-1, keepdims=True)
    var = jnp.mean((out - mu) * (out - mu), axis=-1, keepdims=True)
    out = (out - mu) / jnp.sqrt(var + 1e-5) * g_ref[...] + bb_ref[...]
    if with_res:
        out = out + xres_ref[...]
    o_ref[...] = out


def _node_stage(s, db2, xres, w2t, ln_g, ln_b, with_res):
    n = s.shape[0]
    R = 800
    grid = n // R
    row = lambda i: (i, 0)
    fixed = lambda i: (0, 0)
    return pl.pallas_call(
        functools.partial(_node_body, with_res=with_res),
        grid=(grid,),
        in_specs=[
            pl.BlockSpec((R, HIDDEN), row),
            pl.BlockSpec((R, HIDDEN), row),
            pl.BlockSpec((R, HIDDEN), row),
            pl.BlockSpec((HIDDEN, HIDDEN), fixed),
            pl.BlockSpec((1, HIDDEN), fixed),
            pl.BlockSpec((1, HIDDEN), fixed),
        ],
        out_specs=pl.BlockSpec((R, HIDDEN), row),
        out_shape=jax.ShapeDtypeStruct((n, HIDDEN), jnp.float32),
    )(s, db2, xres, w2t, ln_g, ln_b)


def _head_body(x_ref, m_ref, w1t_ref, b1_ref, w2t_ref, b2_ref, g_ref, bb_ref,
               o_ref):
    x = x_ref[...]
    mu = jnp.mean(x, axis=-1, keepdims=True)
    var = jnp.mean((x - mu) * (x - mu), axis=-1, keepdims=True)
    x = (x - mu) / jnp.sqrt(var + 1e-5) * g_ref[...] + bb_ref[...]
    h = _gelu_exact(jnp.dot(x, w1t_ref[...],
                            preferred_element_type=jnp.float32) + b1_ref[...])
    out = jnp.dot(h, w2t_ref[...],
                  preferred_element_type=jnp.float32) + b2_ref[...]
    o_ref[...] = out * m_ref[...]


def _head(x, mask, w1t, b1, w2t_pad, b2_pad, ln_g, ln_b):
    n = x.shape[0]
    R = 800
    grid = n // R
    row = lambda i: (i, 0)
    fixed = lambda i: (0, 0)
    return pl.pallas_call(
        _head_body,
        grid=(grid,),
        in_specs=[
            pl.BlockSpec((R, HIDDEN), row),
            pl.BlockSpec((R, 8), row),
            pl.BlockSpec((HIDDEN, HIDDEN), fixed),
            pl.BlockSpec((1, HIDDEN), fixed),
            pl.BlockSpec((HIDDEN, 8), fixed),
            pl.BlockSpec((1, 8), fixed),
            pl.BlockSpec((1, HIDDEN), fixed),
            pl.BlockSpec((1, HIDDEN), fixed),
        ],
        out_specs=pl.BlockSpec((R, 8), row),
        out_shape=jax.ShapeDtypeStruct((n, 8), jnp.float32),
    )(x, mask, w1t, b1, w2t_pad, b2_pad, ln_g, ln_b)


def kernel(u_base, pos, velocity_in, airfoil_mask, params):
    B, K, N, C = u_base.shape
    M = max(int(TOP_FRACTION * N), K_NN + 1)
    NB = B * M
    NT = K * NB

    var_per_point = jnp.sum(jnp.var(velocity_in, axis=1, ddof=1), axis=-1)
    mu = jnp.mean(var_per_point, axis=1, keepdims=True)
    sd = jnp.std(var_per_point, axis=1, ddof=1, keepdims=True)
    var_z = (var_per_point - mu) / (sd + 1e-08)
    soft_mask = jax.nn.sigmoid(MASK_SHARPNESS * var_z) * (1.0 - airfoil_mask)
    _, top_idx = jax.lax.top_k(var_per_point, M)

    pos_sel = jnp.take_along_axis(pos, top_idx[:, :, None], axis=1)
    mask_sel = jnp.take_along_axis(soft_mask, top_idx, axis=1)

    pts = lax.stop_gradient(pos_sel)
    knn_local = _sc_knn(pts, B, M).reshape(B, M, K_NN)

    knn_flat = (knn_local + (jnp.arange(B) * M)[:, None, None]).reshape(NB, K_NN)

    pos_cat = pos_sel.reshape(NB, 3)
    rel = pos_cat[knn_flat] - pos_cat[:, None, :]
    dist = jnp.sqrt(jnp.sum(rel * rel, axis=-1, keepdims=True))
    ea = jnp.concatenate([rel, dist], axis=-1).reshape(NB, K_NN * EDGE_DIM)
    ea2 = jnp.concatenate([ea, ea], axis=0)

    v_last = velocity_in[:, -1]
    var_sel = jnp.take_along_axis(var_per_point, top_idx, axis=1)
    ub_sel = jnp.take_along_axis(
        u_base, top_idx[:, None, :, None], axis=2)
    vl_sel = jnp.take_along_axis(v_last, top_idx[:, :, None], axis=1)
    feats = []
    for kk in range(K):
        f = jnp.concatenate(
            [ub_sel[:, kk], vl_sel, pos_sel, var_sel[:, :, None]], axis=-1)
        feats.append(f.reshape(NB, NODE_DIM))
    x = jnp.concatenate(feats, axis=0)

    knn2 = jnp.concatenate([knn_flat, knn_flat + NB], axis=0)
    flat_dst = knn2.reshape(-1)
    deg = jnp.zeros((NT,), jnp.float32).at[flat_dst].add(1.0)

    for i, p in enumerate(params['convs']):
        d_in = NODE_DIM if i == 0 else HIDDEN
        wa = p['w1'][:, :d_in]
        wb = p['w1'][:, d_in:2 * d_in]
        wc = p['w1'][:, 2 * d_in:]
        a = x @ (wa - wb).T
        bv = x @ wb.T
        ag = a[knn2].reshape(NT, K_NN * HIDDEN)
        wcb = jnp.zeros((K_NN * EDGE_DIM, K_NN * HIDDEN), jnp.float32)
        for j in range(K_NN):
            wcb = wcb.at[j * EDGE_DIM:(j + 1) * EDGE_DIM,
                         j * HIDDEN:(j + 1) * HIDDEN].set(wc.T)
        g_edges = _edge_stage(ag, bv, ea2, wcb, p['b1'][None, :])
        s = jnp.zeros((NT, HIDDEN), jnp.float32).at[flat_dst].add(
            g_edges.reshape(NT * K_NN, HIDDEN))
        db2 = deg[:, None] * p['b2'][None, :]
        xres = x if i > 0 else jnp.zeros((NT, HIDDEN), jnp.float32)
        x = _node_stage(s, db2, xres, p['w2'].T,
                        p['ln_g'][None, :], p['ln_b'][None, :],
                        with_res=(i > 0))

    h = params['head']
    mask2 = jnp.concatenate([mask_sel.reshape(NB), mask_sel.reshape(NB)])
    mask_pad = jnp.broadcast_to(mask2[:, None], (NT, 8))
    w2t_pad = jnp.zeros((HIDDEN, 8), jnp.float32).at[:, :3].set(h['w2'].T)
    b2_pad = jnp.zeros((1, 8), jnp.float32).at[0, :3].set(h['b2'])
    delta_pad = _head(x, mask_pad, h['w1'].T, h['b1'][None, :], w2t_pad,
                      b2_pad, h['ln_g'][None, :], h['ln_b'][None, :])
    delta = delta_pad[:, :3]

    delta_kbm = delta.reshape(K, B, M, C).transpose(1, 0, 2, 3)
    bi = jnp.arange(B)[:, None, None]
    ki = jnp.arange(K)[None, :, None]
    delta_full = jnp.zeros((B, K, N, C), u_base.dtype)
    delta_full = delta_full.at[bi, ki, top_idx[:, None, :], :].set(delta_kbm)
    return (u_base + delta_full) * (1.0 - airfoil_mask[:, None, :, None])


# speedup vs baseline: 1.5233x; 1.5233x over previous
import functools
import math

import jax
import jax.numpy as jnp
from jax import lax
from jax.experimental import pallas as pl
from jax.experimental.pallas import tpu as pltpu
from jax.experimental.pallas import tpu_sc as plsc

NODE_DIM = 10
HIDDEN = 64
N_LAYERS = 4
K_NN = 16
TOP_FRACTION = 0.4
MASK_SHARPNESS = 5.0
EDGE_DIM = 4

_INV_SQRT2 = 0.7071067811865476


def _gelu_exact(x):
    return 0.5 * x * (1.0 + lax.erf(x * _INV_SQRT2))


def _knn_body(cols_hbm, out_hbm, px, py, pz, buf, *, M, rows_per_w, nc):
    c = lax.axis_index("c")
    s = lax.axis_index("s")
    wid = s * nc + c
    base = wid * rows_per_w
    b = base // M
    lbase = base - b * M
    cbase = b * 3 * M
    pltpu.sync_copy(cols_hbm.at[pl.ds(cbase, M)], px.at[pl.ds(0, M)])
    pltpu.sync_copy(cols_hbm.at[pl.ds(cbase + M, M)], py.at[pl.ds(0, M)])
    pltpu.sync_copy(cols_hbm.at[pl.ds(cbase + 2 * M, M)], pz.at[pl.ds(0, M)])

    nchunks = M // 16
    inf = jnp.float32(jnp.inf)
    lane = lax.iota(jnp.int32, 16)

    def row_body(r, _):
        il = lbase + r
        xi = px[pl.ds(il, 16)][0]
        yi = py[pl.ds(il, 16)][0]
        zi = pz[pl.ds(il, 16)][0]

        def chunk_body(ci, carry):
            bd, bi, bound = carry
            off = ci * 16
            xs = px[pl.ds(off, 16)]
            ys = py[pl.ds(off, 16)]
            zs = pz[pl.ds(off, 16)]
            dx = xs - xi
            dy = ys - yi
            dz = zs - zi
            d = dx * dx + dy * dy + dz * dz
            idxs = off + lane
            d = jnp.where(idxs == il, inf, d)

            def merge(args):
                bd0, bi0, d0, i0 = args
                cd, cidx = plsc.sort_key_val(d0, i0, descending=True)
                take = cd < bd0
                nd = jnp.where(take, cd, bd0)
                ni = jnp.where(take, cidx, bi0)
                nd, ni = plsc.sort_key_val(nd, ni)
                nb = jnp.broadcast_to(nd[15], (16,))
                return nd, ni, nb

            def keep(args):
                bd0, bi0, _, _ = args
                return bd0, bi0, bound

            cnt = plsc.all_reduce_population_count(d < bound)
            pred = (cnt > 0) if jnp.ndim(cnt) == 0 else (cnt[0] > 0)
            return lax.cond(pred, merge, keep, (bd, bi, d, idxs))

        init = (jnp.full((16,), inf, jnp.float32),
                jnp.zeros((16,), jnp.int32),
                jnp.full((16,), inf, jnp.float32))
        _, bi, _ = lax.fori_loop(0, nchunks, chunk_body, init)
        buf[pl.ds(r * 16, 16)] = bi
        return 0

    lax.fori_loop(0, rows_per_w, row_body, 0)
    pltpu.sync_copy(buf, out_hbm.at[pl.ds(base * K_NN, rows_per_w * K_NN)])


def _sc_knn(pos_sel, B, M):
    info = plsc.get_sparse_core_info()
    nc, ns = info.num_cores, info.num_subcores
    nw = nc * ns
    rows_per_w = (B * M) // nw
    cols = pos_sel.transpose(0, 2, 1).reshape(-1)
    mesh = plsc.VectorSubcoreMesh(core_axis_name="c", subcore_axis_name="s")
    knn = pl.kernel(
        functools.partial(_knn_body, M=M, rows_per_w=rows_per_w, nc=nc),
        out_type=jax.ShapeDtypeStruct((B * M * K_NN,), jnp.int32),
        mesh=mesh,
        compiler_params=pltpu.CompilerParams(needs_layout_passes=False),
        scratch_types=[
            pltpu.VMEM((M + 16,), jnp.float32),
            pltpu.VMEM((M + 16,), jnp.float32),
            pltpu.VMEM((M + 16,), jnp.float32),
            pltpu.VMEM((rows_per_w * K_NN,), jnp.int32),
        ],
    )(cols)
    return knn.reshape(B * M, K_NN)


_GCH = 80
_GNBUF = 4


def _gather_body(tab_hbm, idx_hbm, out_hbm, idx_v, r0, r1, r2, r3,
                 s0, s1, s2, s3, *, rows_per_w, nc):
    c = lax.axis_index("c")
    s = lax.axis_index("s")
    wid = s * nc + c
    base = wid * rows_per_w
    pltpu.sync_copy(idx_hbm.at[pl.ds(base, rows_per_w)], idx_v)
    bufs = (r0, r1, r2, r3)
    sems = (s0, s1, s2, s3)
    nch = rows_per_w // _GCH

    def start(off, b):
        src = tab_hbm.at[idx_v.at[pl.ds(off, _GCH)]]
        pltpu.make_async_copy(src, bufs[b], sems[b]).start()

    def drain(off, b):
        src = tab_hbm.at[idx_v.at[pl.ds(off, _GCH)]]
        pltpu.make_async_copy(src, bufs[b], sems[b]).wait()
        pltpu.sync_copy(bufs[b], out_hbm.at[pl.ds(base + off, _GCH)])

    for b in range(_GNBUF):
        start(b * _GCH, b)

    def group(g, _):
        for b in range(_GNBUF):
            ch = g * _GNBUF + b
            off = pl.multiple_of(ch * _GCH, 16)
            drain(off, b)
            start(pl.multiple_of((ch + _GNBUF) * _GCH, 16), b)
        return 0

    lax.fori_loop(0, nch // _GNBUF - 1, group, 0)
    for b in range(_GNBUF):
        drain((nch - _GNBUF + b) * _GCH, b)


def _sc_gather_rows(tab, idx):
    info = plsc.get_sparse_core_info()
    nc, ns = info.num_cores, info.num_subcores
    nw = nc * ns
    ne = idx.shape[0]
    rows_per_w = ne // nw
    mesh = plsc.VectorSubcoreMesh(core_axis_name="c", subcore_axis_name="s")
    return pl.kernel(
        functools.partial(_gather_body, rows_per_w=rows_per_w, nc=nc),
        out_type=jax.ShapeDtypeStruct((ne, HIDDEN), jnp.float32),
        mesh=mesh,
        compiler_params=pltpu.CompilerParams(needs_layout_passes=False,
                                             use_tc_tiling_on_sc=False),
        scratch_types=(
            [pltpu.VMEM((rows_per_w,), jnp.int32)]
            + [pltpu.VMEM((_GCH, HIDDEN), jnp.float32)] * _GNBUF
            + [pltpu.SemaphoreType.DMA] * _GNBUF
        ),
    )(tab, idx)


def _edge_body(ag_ref, bv_ref, ea_ref, wcb_ref, b1_ref, o_ref):
    call = jnp.dot(ea_ref[...], wcb_ref[...],
                   preferred_element_type=jnp.float32)
    base = ag_ref[...] + call
    bvb = bv_ref[...] + b1_ref[...]
    for j in range(K_NN):
        sl = slice(j * HIDDEN, (j + 1) * HIDDEN)
        o_ref[:, sl] = _gelu_exact(base[:, sl] + bvb)


def _edge_stage(ag, bv, ea, wcb, b1):
    n = ag.shape[0]
    R = 800
    grid = n // R
    row = lambda i: (i, 0)
    fixed = lambda i: (0, 0)
    return pl.pallas_call(
        _edge_body,
        grid=(grid,),
        in_specs=[
            pl.BlockSpec((R, K_NN * HIDDEN), row),
            pl.BlockSpec((R, HIDDEN), row),
            pl.BlockSpec((R, K_NN * EDGE_DIM), row),
            pl.BlockSpec((K_NN * EDGE_DIM, K_NN * HIDDEN), fixed),
            pl.BlockSpec((1, HIDDEN), fixed),
        ],
        out_specs=pl.BlockSpec((R, K_NN * HIDDEN), row),
        out_shape=jax.ShapeDtypeStruct((n, K_NN * HIDDEN), jnp.float32),
    )(ag, bv, ea, wcb, b1)


def _node_body(s_ref, db2_ref, xres_ref, w2t_ref, g_ref, bb_ref, o_ref, *,
               with_res):
    out = jnp.dot(s_ref[...], w2t_ref[...],
                  preferred_element_type=jnp.float32) + db2_ref[...]
    mu = jnp.mean(out, axis=-1, keepdims=True)
    var = jnp.mean((out - mu) * (out - mu), axis=-1, keepdims=True)
    out = (out - mu) / jnp.sqrt(var + 1e-5) * g_ref[...] + bb_ref[...]
    if with_res:
        out = out + xres_ref[...]
    o_ref[...] = out


def _node_stage(s, db2, xres, w2t, ln_g, ln_b, with_res):
    n = s.shape[0]
    R = 800
    grid = n // R
    row = lambda i: (i, 0)
    fixed = lambda i: (0, 0)
    return pl.pallas_call(
        functools.partial(_node_body, with_res=with_res),
        grid=(grid,),
        in_specs=[
            pl.BlockSpec((R, HIDDEN), row),
            pl.BlockSpec((R, HIDDEN), row),
            pl.BlockSpec((R, HIDDEN), row),
            pl.BlockSpec((HIDDEN, HIDDEN), fixed),
            pl.BlockSpec((1, HIDDEN), fixed),
            pl.BlockSpec((1, HIDDEN), fixed),
        ],
        out_specs=pl.BlockSpec((R, HIDDEN), row),
        out_shape=jax.ShapeDtypeStruct((n, HIDDEN), jnp.float32),
    )(s, db2, xres, w2t, ln_g, ln_b)


def _head_body(x_ref, m_ref, w1t_ref, b1_ref, w2t_ref, b2_ref, g_ref, bb_ref,
               o_ref):
    x = x_ref[...]
    mu = jnp.mean(x, axis=-1, keepdims=True)
    var = jnp.mean((x - mu) * (x - mu), axis=-1, keepdims=True)
    x = (x - mu) / jnp.sqrt(var + 1e-5) * g_ref[...] + bb_ref[...]
    h = _gelu_exact(jnp.dot(x, w1t_ref[...],
                            preferred_element_type=jnp.float32) + b1_ref[...])
    out = jnp.dot(h, w2t_ref[...],
                  preferred_element_type=jnp.float32) + b2_ref[...]
    o_ref[...] = out * m_ref[...]


def _head(x, mask, w1t, b1, w2t_pad, b2_pad, ln_g, ln_b):
    n = x.shape[0]
    R = 800
    grid = n // R
    row = lambda i: (i, 0)
    fixed = lambda i: (0, 0)
    return pl.pallas_call(
        _head_body,
        grid=(grid,),
        in_specs=[
            pl.BlockSpec((R, HIDDEN), row),
            pl.BlockSpec((R, 8), row),
            pl.BlockSpec((HIDDEN, HIDDEN), fixed),
            pl.BlockSpec((1, HIDDEN), fixed),
            pl.BlockSpec((HIDDEN, 8), fixed),
            pl.BlockSpec((1, 8), fixed),
            pl.BlockSpec((1, HIDDEN), fixed),
            pl.BlockSpec((1, HIDDEN), fixed),
        ],
        out_specs=pl.BlockSpec((R, 8), row),
        out_shape=jax.ShapeDtypeStruct((n, 8), jnp.float32),
    )(x, mask, w1t, b1, w2t_pad, b2_pad, ln_g, ln_b)


def kernel(u_base, pos, velocity_in, airfoil_mask, params):
    B, K, N, C = u_base.shape
    M = max(int(TOP_FRACTION * N), K_NN + 1)
    NB = B * M
    NT = K * NB

    var_per_point = jnp.sum(jnp.var(velocity_in, axis=1, ddof=1), axis=-1)
    mu = jnp.mean(var_per_point, axis=1, keepdims=True)
    sd = jnp.std(var_per_point, axis=1, ddof=1, keepdims=True)
    var_z = (var_per_point - mu) / (sd + 1e-08)
    soft_mask = jax.nn.sigmoid(MASK_SHARPNESS * var_z) * (1.0 - airfoil_mask)
    _, top_idx = jax.lax.top_k(var_per_point, M)

    pos_sel = jnp.take_along_axis(pos, top_idx[:, :, None], axis=1)
    mask_sel = jnp.take_along_axis(soft_mask, top_idx, axis=1)

    pts = lax.stop_gradient(pos_sel)
    knn_local = _sc_knn(pts, B, M).reshape(B, M, K_NN)

    knn_flat = (knn_local + (jnp.arange(B) * M)[:, None, None]).reshape(NB, K_NN)

    pos_cat = pos_sel.reshape(NB, 3)
    rel = pos_cat[knn_flat] - pos_cat[:, None, :]
    dist = jnp.sqrt(jnp.sum(rel * rel, axis=-1, keepdims=True))
    ea = jnp.concatenate([rel, dist], axis=-1).reshape(NB, K_NN * EDGE_DIM)
    ea2 = jnp.concatenate([ea, ea], axis=0)

    v_last = velocity_in[:, -1]
    var_sel = jnp.take_along_axis(var_per_point, top_idx, axis=1)
    ub_sel = jnp.take_along_axis(
        u_base, top_idx[:, None, :, None], axis=2)
    vl_sel = jnp.take_along_axis(v_last, top_idx[:, :, None], axis=1)
    feats = []
    for kk in range(K):
        f = jnp.concatenate(
            [ub_sel[:, kk], vl_sel, pos_sel, var_sel[:, :, None]], axis=-1)
        feats.append(f.reshape(NB, NODE_DIM))
    x = jnp.concatenate(feats, axis=0)

    knn2 = jnp.concatenate([knn_flat, knn_flat + NB], axis=0)
    flat_dst = knn2.reshape(-1)
    deg = jnp.zeros((NT,), jnp.float32).at[flat_dst].add(1.0)

    for i, p in enumerate(params['convs']):
        d_in = NODE_DIM if i == 0 else HIDDEN
        wa = p['w1'][:, :d_in]
        wb = p['w1'][:, d_in:2 * d_in]
        wc = p['w1'][:, 2 * d_in:]
        a = x @ (wa - wb).T
        bv = x @ wb.T
        ag = _sc_gather_rows(a, flat_dst).reshape(NT, K_NN * HIDDEN)
        wcb = jnp.zeros((K_NN * EDGE_DIM, K_NN * HIDDEN), jnp.float32)
        for j in range(K_NN):
            wcb = wcb.at[j * EDGE_DIM:(j + 1) * EDGE_DIM,
                         j * HIDDEN:(j + 1) * HIDDEN].set(wc.T)
        g_edges = _edge_stage(ag, bv, ea2, wcb, p['b1'][None, :])
        s = jnp.zeros((NT, HIDDEN), jnp.float32).at[flat_dst].add(
            g_edges.reshape(NT * K_NN, HIDDEN))
        db2 = deg[:, None] * p['b2'][None, :]
        xres = x if i > 0 else jnp.zeros((NT, HIDDEN), jnp.float32)
        x = _node_stage(s, db2, xres, p['w2'].T,
                        p['ln_g'][None, :], p['ln_b'][None, :],
                        with_res=(i > 0))

    h = params['head']
    mask2 = jnp.concatenate([mask_sel.reshape(NB), mask_sel.reshape(NB)])
    mask_pad = jnp.broadcast_to(mask2[:, None], (NT, 8))
    w2t_pad = jnp.zeros((HIDDEN, 8), jnp.float32).at[:, :3].set(h['w2'].T)
    b2_pad = jnp.zeros((1, 8), jnp.float32).at[0, :3].set(h['b2'])
    delta_pad = _head(x, mask_pad, h['w1'].T, h['b1'][None, :], w2t_pad,
                      b2_pad, h['ln_g'][None, :], h['ln_b'][None, :])
    delta = delta_pad[:, :3]

    delta_kbm = delta.reshape(K, B, M, C).transpose(1, 0, 2, 3)
    bi = jnp.arange(B)[:, None, None]
    ki = jnp.arange(K)[None, :, None]
    delta_full = jnp.zeros((B, K, N, C), u_base.dtype)
    delta_full = delta_full.at[bi, ki, top_idx[:, None, :], :].set(delta_kbm)
    return (u_base + delta_full) * (1.0 - airfoil_mask[:, None, :, None])


# speedup vs baseline: 1.8191x; 1.1941x over previous
import functools
import math

import jax
import jax.numpy as jnp
from jax import lax
from jax.experimental import pallas as pl
from jax.experimental.pallas import tpu as pltpu
from jax.experimental.pallas import tpu_sc as plsc

NODE_DIM = 10
HIDDEN = 64
N_LAYERS = 4
K_NN = 16
TOP_FRACTION = 0.4
MASK_SHARPNESS = 5.0
EDGE_DIM = 4

_INV_SQRT2 = 0.7071067811865476


def _gelu_exact(x):
    return 0.5 * x * (1.0 + lax.erf(x * _INV_SQRT2))


def _knn_body(cols_hbm, out_hbm, px, py, pz, buf, *, M, rows_per_w, nc):
    c = lax.axis_index("c")
    s = lax.axis_index("s")
    wid = s * nc + c
    base = wid * rows_per_w
    b = base // M
    lbase = base - b * M
    cbase = b * 3 * M
    pltpu.sync_copy(cols_hbm.at[pl.ds(cbase, M)], px.at[pl.ds(0, M)])
    pltpu.sync_copy(cols_hbm.at[pl.ds(cbase + M, M)], py.at[pl.ds(0, M)])
    pltpu.sync_copy(cols_hbm.at[pl.ds(cbase + 2 * M, M)], pz.at[pl.ds(0, M)])

    nchunks = M // 16
    inf = jnp.float32(jnp.inf)
    lane = lax.iota(jnp.int32, 16)

    unroll = 5

    def row_body(r, _):
        il = lbase + r
        xi = px[pl.ds(il, 16)][0]
        yi = py[pl.ds(il, 16)][0]
        zi = pz[pl.ds(il, 16)][0]

        def group_body(gi, carry):
            bd, bi, bound = carry
            ds_, is_ = [], []
            hit = None
            for t in range(unroll):
                off = (gi * unroll + t) * 16
                xs = px[pl.ds(off, 16)]
                ys = py[pl.ds(off, 16)]
                zs = pz[pl.ds(off, 16)]
                dx = xs - xi
                dy = ys - yi
                dz = zs - zi
                d = dx * dx + dy * dy + dz * dz
                idxs = off + lane
                d = jnp.where(idxs == il, inf, d)
                m = d < bound
                hit = m if hit is None else (hit | m)
                ds_.append(d)
                is_.append(idxs)

            def merge(args):
                bd0, bi0 = args[0], args[1]
                for t in range(unroll):
                    d0, i0 = args[2 + 2 * t], args[3 + 2 * t]
                    cd, cidx = plsc.sort_key_val(d0, i0, descending=True)
                    take = cd < bd0
                    nd = jnp.where(take, cd, bd0)
                    ni = jnp.where(take, cidx, bi0)
                    bd0, bi0 = plsc.sort_key_val(nd, ni)
                nb = jnp.broadcast_to(bd0[15], (16,))
                return bd0, bi0, nb

            def keep(args):
                return args[0], args[1], bound

            cnt = plsc.all_reduce_population_count(hit)
            pred = (cnt > 0) if jnp.ndim(cnt) == 0 else (cnt[0] > 0)
            flat = (bd, bi) + tuple(
                v for t in range(unroll) for v in (ds_[t], is_[t]))
            return lax.cond(pred, merge, keep, flat)

        init = (jnp.full((16,), inf, jnp.float32),
                jnp.zeros((16,), jnp.int32),
                jnp.full((16,), inf, jnp.float32))
        _, bi, _ = lax.fori_loop(0, nchunks // unroll, group_body, init)
        buf[pl.ds(r * 16, 16)] = bi
        return 0

    lax.fori_loop(0, rows_per_w, row_body, 0)
    pltpu.sync_copy(buf, out_hbm.at[pl.ds(base * K_NN, rows_per_w * K_NN)])


def _sc_knn(pos_sel, B, M):
    info = plsc.get_sparse_core_info()
    nc, ns = info.num_cores, info.num_subcores
    nw = nc * ns
    rows_per_w = (B * M) // nw
    cols = pos_sel.transpose(0, 2, 1).reshape(-1)
    mesh = plsc.VectorSubcoreMesh(core_axis_name="c", subcore_axis_name="s")
    knn = pl.kernel(
        functools.partial(_knn_body, M=M, rows_per_w=rows_per_w, nc=nc),
        out_type=jax.ShapeDtypeStruct((B * M * K_NN,), jnp.int32),
        mesh=mesh,
        compiler_params=pltpu.CompilerParams(needs_layout_passes=False),
        scratch_types=[
            pltpu.VMEM((M + 16,), jnp.float32),
            pltpu.VMEM((M + 16,), jnp.float32),
            pltpu.VMEM((M + 16,), jnp.float32),
            pltpu.VMEM((rows_per_w * K_NN,), jnp.int32),
        ],
    )(cols)
    return knn.reshape(B * M, K_NN)


_GCH = 80
_GNBUF = 4


def _gather_body(tab_hbm, idx_hbm, out_hbm, idx_v, r0, r1, r2, r3,
                 s0, s1, s2, s3, *, rows_per_w, nc):
    c = lax.axis_index("c")
    s = lax.axis_index("s")
    wid = s * nc + c
    base = wid * rows_per_w
    pltpu.sync_copy(idx_hbm.at[pl.ds(base, rows_per_w)], idx_v)
    bufs = (r0, r1, r2, r3)
    sems = (s0, s1, s2, s3)
    nch = rows_per_w // _GCH

    def start(off, b):
        src = tab_hbm.at[idx_v.at[pl.ds(off, _GCH)]]
        pltpu.make_async_copy(src, bufs[b], sems[b]).start()

    def drain(off, b):
        src = tab_hbm.at[idx_v.at[pl.ds(off, _GCH)]]
        pltpu.make_async_copy(src, bufs[b], sems[b]).wait()
        pltpu.sync_copy(bufs[b], out_hbm.at[pl.ds(base + off, _GCH)])

    for b in range(_GNBUF):
        start(b * _GCH, b)

    def group(g, _):
        for b in range(_GNBUF):
            ch = g * _GNBUF + b
            off = pl.multiple_of(ch * _GCH, 16)
            drain(off, b)
            start(pl.multiple_of((ch + _GNBUF) * _GCH, 16), b)
        return 0

    lax.fori_loop(0, nch // _GNBUF - 1, group, 0)
    for b in range(_GNBUF):
        drain((nch - _GNBUF + b) * _GCH, b)


def _sc_gather_rows(tab, idx):
    info = plsc.get_sparse_core_info()
    nc, ns = info.num_cores, info.num_subcores
    nw = nc * ns
    ne = idx.shape[0]
    rows_per_w = ne // nw
    mesh = plsc.VectorSubcoreMesh(core_axis_name="c", subcore_axis_name="s")
    return pl.kernel(
        functools.partial(_gather_body, rows_per_w=rows_per_w, nc=nc),
        out_type=jax.ShapeDtypeStruct((ne, HIDDEN), jnp.float32),
        mesh=mesh,
        compiler_params=pltpu.CompilerParams(needs_layout_passes=False,
                                             use_tc_tiling_on_sc=False),
        scratch_types=(
            [pltpu.VMEM((rows_per_w,), jnp.int32)]
            + [pltpu.VMEM((_GCH, HIDDEN), jnp.float32)] * _GNBUF
            + [pltpu.SemaphoreType.DMA] * _GNBUF
        ),
    )(tab, idx)


def _edge_body(ag_ref, bv_ref, ea_ref, wcb_ref, b1_ref, o_ref):
    call = jnp.dot(ea_ref[...], wcb_ref[...],
                   preferred_element_type=jnp.float32)
    base = ag_ref[...] + call
    bvb = bv_ref[...] + b1_ref[...]
    for j in range(K_NN):
        sl = slice(j * HIDDEN, (j + 1) * HIDDEN)
        o_ref[:, sl] = _gelu_exact(base[:, sl] + bvb)


def _edge_stage(ag, bv, ea, wcb, b1):
    n = ag.shape[0]
    R = 800
    grid = n // R
    row = lambda i: (i, 0)
    fixed = lambda i: (0, 0)
    return pl.pallas_call(
        _edge_body,
        grid=(grid,),
        in_specs=[
            pl.BlockSpec((R, K_NN * HIDDEN), row),
            pl.BlockSpec((R, HIDDEN), row),
            pl.BlockSpec((R, K_NN * EDGE_DIM), row),
            pl.BlockSpec((K_NN * EDGE_DIM, K_NN * HIDDEN), fixed),
            pl.BlockSpec((1, HIDDEN), fixed),
        ],
        out_specs=pl.BlockSpec((R, K_NN * HIDDEN), row),
        out_shape=jax.ShapeDtypeStruct((n, K_NN * HIDDEN), jnp.float32),
    )(ag, bv, ea, wcb, b1)


def _node_body(s_ref, db2_ref, xres_ref, w2t_ref, g_ref, bb_ref,
               o_ref, *, with_res):
    out = jnp.dot(s_ref[...], w2t_ref[...],
                  preferred_element_type=jnp.float32) + db2_ref[...]
    mu = jnp.mean(out, axis=-1, keepdims=True)
    var = jnp.mean((out - mu) * (out - mu), axis=-1, keepdims=True)
    out = (out - mu) / jnp.sqrt(var + 1e-5) * g_ref[...] + bb_ref[...]
    if with_res:
        out = out + xres_ref[...]
    o_ref[...] = out


def _node_stage(s, db2, xres, w2t, ln_g, ln_b, with_res):
    n = s.shape[0]
    R = 800
    grid = n // R
    row = lambda i: (i, 0)
    fixed = lambda i: (0, 0)
    return pl.pallas_call(
        functools.partial(_node_body, with_res=with_res),
        grid=(grid,),
        in_specs=[
            pl.BlockSpec((R, HIDDEN), row),
            pl.BlockSpec((R, HIDDEN), row),
            pl.BlockSpec((R, HIDDEN), row),
            pl.BlockSpec((HIDDEN, HIDDEN), fixed),
            pl.BlockSpec((1, HIDDEN), fixed),
            pl.BlockSpec((1, HIDDEN), fixed),
        ],
        out_specs=pl.BlockSpec((R, HIDDEN), row),
        out_shape=jax.ShapeDtypeStruct((n, HIDDEN), jnp.float32),
    )(s, db2, xres, w2t, ln_g, ln_b)


def _head_body(x_ref, m_ref, w1t_ref, b1_ref, w2t_ref, b2_ref, g_ref, bb_ref,
               o_ref):
    x = x_ref[...]
    mu = jnp.mean(x, axis=-1, keepdims=True)
    var = jnp.mean((x - mu) * (x - mu), axis=-1, keepdims=True)
    x = (x - mu) / jnp.sqrt(var + 1e-5) * g_ref[...] + bb_ref[...]
    h = _gelu_exact(jnp.dot(x, w1t_ref[...],
                            preferred_element_type=jnp.float32) + b1_ref[...])
    out = jnp.dot(h, w2t_ref[...],
                  preferred_element_type=jnp.float32) + b2_ref[...]
    o_ref[...] = out * m_ref[...]


def _head(x, mask, w1t, b1, w2t_pad, b2_pad, ln_g, ln_b):
    n = x.shape[0]
    R = 800
    grid = n // R
    row = lambda i: (i, 0)
    fixed = lambda i: (0, 0)
    return pl.pallas_call(
        _head_body,
        grid=(grid,),
        in_specs=[
            pl.BlockSpec((R, HIDDEN), row),
            pl.BlockSpec((R, 8), row),
            pl.BlockSpec((HIDDEN, HIDDEN), fixed),
            pl.BlockSpec((1, HIDDEN), fixed),
            pl.BlockSpec((HIDDEN, 8), fixed),
            pl.BlockSpec((1, 8), fixed),
            pl.BlockSpec((1, HIDDEN), fixed),
            pl.BlockSpec((1, HIDDEN), fixed),
        ],
        out_specs=pl.BlockSpec((R, 8), row),
        out_shape=jax.ShapeDtypeStruct((n, 8), jnp.float32),
    )(x, mask, w1t, b1, w2t_pad, b2_pad, ln_g, ln_b)


def kernel(u_base, pos, velocity_in, airfoil_mask, params):
    B, K, N, C = u_base.shape
    M = max(int(TOP_FRACTION * N), K_NN + 1)
    NB = B * M
    NT = K * NB

    var_per_point = jnp.sum(jnp.var(velocity_in, axis=1, ddof=1), axis=-1)
    mu = jnp.mean(var_per_point, axis=1, keepdims=True)
    sd = jnp.std(var_per_point, axis=1, ddof=1, keepdims=True)
    var_z = (var_per_point - mu) / (sd + 1e-08)
    soft_mask = jax.nn.sigmoid(MASK_SHARPNESS * var_z) * (1.0 - airfoil_mask)
    _, top_idx = jax.lax.top_k(var_per_point, M)

    pos_sel = jnp.take_along_axis(pos, top_idx[:, :, None], axis=1)
    mask_sel = jnp.take_along_axis(soft_mask, top_idx, axis=1)

    pts = lax.stop_gradient(pos_sel)
    knn_local = _sc_knn(pts, B, M).reshape(B, M, K_NN)

    knn_flat = (knn_local + (jnp.arange(B) * M)[:, None, None]).reshape(NB, K_NN)

    pos_cat = pos_sel.reshape(NB, 3)
    rel = pos_cat[knn_flat] - pos_cat[:, None, :]
    dist = jnp.sqrt(jnp.sum(rel * rel, axis=-1, keepdims=True))
    ea = jnp.concatenate([rel, dist], axis=-1).reshape(NB, K_NN * EDGE_DIM)
    ea2 = jnp.concatenate([ea, ea], axis=0)

    v_last = velocity_in[:, -1]
    var_sel = jnp.take_along_axis(var_per_point, top_idx, axis=1)
    ub_sel = jnp.take_along_axis(
        u_base, top_idx[:, None, :, None], axis=2)
    vl_sel = jnp.take_along_axis(v_last, top_idx[:, :, None], axis=1)
    feats = []
    for kk in range(K):
        f = jnp.concatenate(
            [ub_sel[:, kk], vl_sel, pos_sel, var_sel[:, :, None]], axis=-1)
        feats.append(f.reshape(NB, NODE_DIM))
    x = jnp.concatenate(feats, axis=0)

    knn2 = jnp.concatenate([knn_flat, knn_flat + NB], axis=0)
    flat_dst = knn2.reshape(-1)
    deg = jnp.zeros((NT,), jnp.float32).at[flat_dst].add(1.0)

    for i, p in enumerate(params['convs']):
        d_in = NODE_DIM if i == 0 else HIDDEN
        wa = p['w1'][:, :d_in]
        wb = p['w1'][:, d_in:2 * d_in]
        wc = p['w1'][:, 2 * d_in:]
        a = x @ (wa - wb).T
        bv = x @ wb.T
        ag = _sc_gather_rows(a, flat_dst).reshape(NT, K_NN * HIDDEN)
        wcb = jnp.zeros((K_NN * EDGE_DIM, K_NN * HIDDEN), jnp.float32)
        for j in range(K_NN):
            wcb = wcb.at[j * EDGE_DIM:(j + 1) * EDGE_DIM,
                         j * HIDDEN:(j + 1) * HIDDEN].set(wc.T)
        g_edges = _edge_stage(ag, bv, ea2, wcb, p['b1'][None, :])
        s = jnp.zeros((NT, HIDDEN), jnp.float32).at[flat_dst].add(
            g_edges.reshape(NT * K_NN, HIDDEN))
        db2 = deg[:, None] * p['b2'][None, :]
        xres = x if i > 0 else jnp.zeros((NT, HIDDEN), jnp.float32)
        x = _node_stage(s, db2, xres, p['w2'].T,
                        p['ln_g'][None, :], p['ln_b'][None, :],
                        with_res=(i > 0))

    h = params['head']
    mask2 = jnp.concatenate([mask_sel.reshape(NB), mask_sel.reshape(NB)])
    mask_pad = jnp.broadcast_to(mask2[:, None], (NT, 8))
    w2t_pad = jnp.zeros((HIDDEN, 8), jnp.float32).at[:, :3].set(h['w2'].T)
    b2_pad = jnp.zeros((1, 8), jnp.float32).at[0, :3].set(h['b2'])
    delta_pad = _head(x, mask_pad, h['w1'].T, h['b1'][None, :], w2t_pad,
                      b2_pad, h['ln_g'][None, :], h['ln_b'][None, :])
    delta = delta_pad[:, :3]

    delta_kbm = delta.reshape(K, B, M, C).transpose(1, 0, 2, 3)
    bi = jnp.arange(B)[:, None, None]
    ki = jnp.arange(K)[None, :, None]
    delta_full = jnp.zeros((B, K, N, C), u_base.dtype)
    delta_full = delta_full.at[bi, ki, top_idx[:, None, :], :].set(delta_kbm)
    return (u_base + delta_full) * (1.0 - airfoil_mask[:, None, :, None])


# speedup vs baseline: 1.8510x; 1.0176x over previous
import functools
import math

import jax
import jax.numpy as jnp
from jax import lax
from jax.experimental import pallas as pl
from jax.experimental.pallas import tpu as pltpu
from jax.experimental.pallas import tpu_sc as plsc

NODE_DIM = 10
HIDDEN = 64
N_LAYERS = 4
K_NN = 16
TOP_FRACTION = 0.4
MASK_SHARPNESS = 5.0
EDGE_DIM = 4

_INV_SQRT2 = 0.7071067811865476


def _gelu_exact(x):
    return 0.5 * x * (1.0 + lax.erf(x * _INV_SQRT2))


def _knn_body(cols_hbm, out_hbm, px, py, pz, buf, *, M, rows_per_w, nc):
    c = lax.axis_index("c")
    s = lax.axis_index("s")
    wid = s * nc + c
    base = wid * rows_per_w
    b = base // M
    lbase = base - b * M
    cbase = b * 3 * M
    pltpu.sync_copy(cols_hbm.at[pl.ds(cbase, M)], px.at[pl.ds(0, M)])
    pltpu.sync_copy(cols_hbm.at[pl.ds(cbase + M, M)], py.at[pl.ds(0, M)])
    pltpu.sync_copy(cols_hbm.at[pl.ds(cbase + 2 * M, M)], pz.at[pl.ds(0, M)])

    nchunks = M // 16
    inf = jnp.float32(jnp.inf)
    lane = lax.iota(jnp.int32, 16)

    unroll = 10

    def row_body(r, _):
        il = lbase + r
        xi = px[pl.ds(il, 16)][0]
        yi = py[pl.ds(il, 16)][0]
        zi = pz[pl.ds(il, 16)][0]

        def group_body(gi, carry):
            bd, bi, bound = carry
            ds_, is_ = [], []
            hit = None
            for t in range(unroll):
                off = (gi * unroll + t) * 16
                xs = px[pl.ds(off, 16)]
                ys = py[pl.ds(off, 16)]
                zs = pz[pl.ds(off, 16)]
                dx = xs - xi
                dy = ys - yi
                dz = zs - zi
                d = dx * dx + dy * dy + dz * dz
                idxs = off + lane
                d = jnp.where(idxs == il, inf, d)
                m = d < bound
                hit = m if hit is None else (hit | m)
                ds_.append(d)
                is_.append(idxs)

            def merge(args):
                bd0, bi0 = args[0], args[1]
                for t in range(unroll):
                    d0, i0 = args[2 + 2 * t], args[3 + 2 * t]
                    cd, cidx = plsc.sort_key_val(d0, i0, descending=True)
                    take = cd < bd0
                    nd = jnp.where(take, cd, bd0)
                    ni = jnp.where(take, cidx, bi0)
                    bd0, bi0 = plsc.sort_key_val(nd, ni)
                nb = jnp.broadcast_to(bd0[15], (16,))
                return bd0, bi0, nb

            def keep(args):
                return args[0], args[1], bound

            cnt = plsc.all_reduce_population_count(hit)
            pred = (cnt > 0) if jnp.ndim(cnt) == 0 else (cnt[0] > 0)
            flat = (bd, bi) + tuple(
                v for t in range(unroll) for v in (ds_[t], is_[t]))
            return lax.cond(pred, merge, keep, flat)

        init = (jnp.full((16,), inf, jnp.float32),
                jnp.zeros((16,), jnp.int32),
                jnp.full((16,), inf, jnp.float32))
        _, bi, _ = lax.fori_loop(0, nchunks // unroll, group_body, init)
        buf[pl.ds(r * 16, 16)] = bi
        return 0

    lax.fori_loop(0, rows_per_w, row_body, 0)
    pltpu.sync_copy(buf, out_hbm.at[pl.ds(base * K_NN, rows_per_w * K_NN)])


def _sc_knn(pos_sel, B, M):
    info = plsc.get_sparse_core_info()
    nc, ns = info.num_cores, info.num_subcores
    nw = nc * ns
    rows_per_w = (B * M) // nw
    cols = pos_sel.transpose(0, 2, 1).reshape(-1)
    mesh = plsc.VectorSubcoreMesh(core_axis_name="c", subcore_axis_name="s")
    knn = pl.kernel(
        functools.partial(_knn_body, M=M, rows_per_w=rows_per_w, nc=nc),
        out_type=jax.ShapeDtypeStruct((B * M * K_NN,), jnp.int32),
        mesh=mesh,
        compiler_params=pltpu.CompilerParams(needs_layout_passes=False),
        scratch_types=[
            pltpu.VMEM((M + 16,), jnp.float32),
            pltpu.VMEM((M + 16,), jnp.float32),
            pltpu.VMEM((M + 16,), jnp.float32),
            pltpu.VMEM((rows_per_w * K_NN,), jnp.int32),
        ],
    )(cols)
    return knn.reshape(B * M, K_NN)


_GCH = 80
_GNBUF = 4


def _gather_body(tab_hbm, idx_hbm, out_hbm, idx_v, r0, r1, r2, r3,
                 s0, s1, s2, s3, *, rows_per_w, nc):
    c = lax.axis_index("c")
    s = lax.axis_index("s")
    wid = s * nc + c
    base = wid * rows_per_w
    pltpu.sync_copy(idx_hbm.at[pl.ds(base, rows_per_w)], idx_v)
    bufs = (r0, r1, r2, r3)
    sems = (s0, s1, s2, s3)
    nch = rows_per_w // _GCH

    def start(off, b):
        src = tab_hbm.at[idx_v.at[pl.ds(off, _GCH)]]
        pltpu.make_async_copy(src, bufs[b], sems[b]).start()

    def drain(off, b):
        src = tab_hbm.at[idx_v.at[pl.ds(off, _GCH)]]
        pltpu.make_async_copy(src, bufs[b], sems[b]).wait()
        pltpu.sync_copy(bufs[b], out_hbm.at[pl.ds(base + off, _GCH)])

    for b in range(_GNBUF):
        start(b * _GCH, b)

    def group(g, _):
        for b in range(_GNBUF):
            ch = g * _GNBUF + b
            off = pl.multiple_of(ch * _GCH, 16)
            drain(off, b)
            start(pl.multiple_of((ch + _GNBUF) * _GCH, 16), b)
        return 0

    lax.fori_loop(0, nch // _GNBUF - 1, group, 0)
    for b in range(_GNBUF):
        drain((nch - _GNBUF + b) * _GCH, b)


def _sc_gather_rows(tab, idx):
    info = plsc.get_sparse_core_info()
    nc, ns = info.num_cores, info.num_subcores
    nw = nc * ns
    ne = idx.shape[0]
    rows_per_w = ne // nw
    mesh = plsc.VectorSubcoreMesh(core_axis_name="c", subcore_axis_name="s")
    return pl.kernel(
        functools.partial(_gather_body, rows_per_w=rows_per_w, nc=nc),
        out_type=jax.ShapeDtypeStruct((ne, HIDDEN), jnp.float32),
        mesh=mesh,
        compiler_params=pltpu.CompilerParams(needs_layout_passes=False,
                                             use_tc_tiling_on_sc=False),
        scratch_types=(
            [pltpu.VMEM((rows_per_w,), jnp.int32)]
            + [pltpu.VMEM((_GCH, HIDDEN), jnp.float32)] * _GNBUF
            + [pltpu.SemaphoreType.DMA] * _GNBUF
        ),
    )(tab, idx)


def _edge_body(ag_ref, bv_ref, ea_ref, wcb_ref, b1_ref, o_ref):
    call = jnp.dot(ea_ref[...], wcb_ref[...],
                   preferred_element_type=jnp.float32)
    base = ag_ref[...] + call
    bvb = bv_ref[...] + b1_ref[...]
    for j in range(K_NN):
        sl = slice(j * HIDDEN, (j + 1) * HIDDEN)
        o_ref[:, sl] = _gelu_exact(base[:, sl] + bvb)


def _edge_stage(ag, bv, ea, wcb, b1):
    n = ag.shape[0]
    R = 800
    grid = n // R
    row = lambda i: (i, 0)
    fixed = lambda i: (0, 0)
    return pl.pallas_call(
        _edge_body,
        grid=(grid,),
        in_specs=[
            pl.BlockSpec((R, K_NN * HIDDEN), row),
            pl.BlockSpec((R, HIDDEN), row),
            pl.BlockSpec((R, K_NN * EDGE_DIM), row),
            pl.BlockSpec((K_NN * EDGE_DIM, K_NN * HIDDEN), fixed),
            pl.BlockSpec((1, HIDDEN), fixed),
        ],
        out_specs=pl.BlockSpec((R, K_NN * HIDDEN), row),
        out_shape=jax.ShapeDtypeStruct((n, K_NN * HIDDEN), jnp.float32),
    )(ag, bv, ea, wcb, b1)


def _node_body(s_ref, db2_ref, xres_ref, w2t_ref, g_ref, bb_ref,
               o_ref, *, with_res):
    out = jnp.dot(s_ref[...], w2t_ref[...],
                  preferred_element_type=jnp.float32) + db2_ref[...]
    mu = jnp.mean(out, axis=-1, keepdims=True)
    var = jnp.mean((out - mu) * (out - mu), axis=-1, keepdims=True)
    out = (out - mu) / jnp.sqrt(var + 1e-5) * g_ref[...] + bb_ref[...]
    if with_res:
        out = out + xres_ref[...]
    o_ref[...] = out


def _node_stage(s, db2, xres, w2t, ln_g, ln_b, with_res):
    n = s.shape[0]
    R = 800
    grid = n // R
    row = lambda i: (i, 0)
    fixed = lambda i: (0, 0)
    return pl.pallas_call(
        functools.partial(_node_body, with_res=with_res),
        grid=(grid,),
        in_specs=[
            pl.BlockSpec((R, HIDDEN), row),
            pl.BlockSpec((R, HIDDEN), row),
            pl.BlockSpec((R, HIDDEN), row),
            pl.BlockSpec((HIDDEN, HIDDEN), fixed),
            pl.BlockSpec((1, HIDDEN), fixed),
            pl.BlockSpec((1, HIDDEN), fixed),
        ],
        out_specs=pl.BlockSpec((R, HIDDEN), row),
        out_shape=jax.ShapeDtypeStruct((n, HIDDEN), jnp.float32),
    )(s, db2, xres, w2t, ln_g, ln_b)


def _head_body(x_ref, m_ref, w1t_ref, b1_ref, w2t_ref, b2_ref, g_ref, bb_ref,
               o_ref):
    x = x_ref[...]
    mu = jnp.mean(x, axis=-1, keepdims=True)
    var = jnp.mean((x - mu) * (x - mu), axis=-1, keepdims=True)
    x = (x - mu) / jnp.sqrt(var + 1e-5) * g_ref[...] + bb_ref[...]
    h = _gelu_exact(jnp.dot(x, w1t_ref[...],
                            preferred_element_type=jnp.float32) + b1_ref[...])
    out = jnp.dot(h, w2t_ref[...],
                  preferred_element_type=jnp.float32) + b2_ref[...]
    o_ref[...] = out * m_ref[...]


def _head(x, mask, w1t, b1, w2t_pad, b2_pad, ln_g, ln_b):
    n = x.shape[0]
    R = 800
    grid = n // R
    row = lambda i: (i, 0)
    fixed = lambda i: (0, 0)
    return pl.pallas_call(
        _head_body,
        grid=(grid,),
        in_specs=[
            pl.BlockSpec((R, HIDDEN), row),
            pl.BlockSpec((R, 8), row),
            pl.BlockSpec((HIDDEN, HIDDEN), fixed),
            pl.BlockSpec((1, HIDDEN), fixed),
            pl.BlockSpec((HIDDEN, 8), fixed),
            pl.BlockSpec((1, 8), fixed),
            pl.BlockSpec((1, HIDDEN), fixed),
            pl.BlockSpec((1, HIDDEN), fixed),
        ],
        out_specs=pl.BlockSpec((R, 8), row),
        out_shape=jax.ShapeDtypeStruct((n, 8), jnp.float32),
    )(x, mask, w1t, b1, w2t_pad, b2_pad, ln_g, ln_b)


def kernel(u_base, pos, velocity_in, airfoil_mask, params):
    B, K, N, C = u_base.shape
    M = max(int(TOP_FRACTION * N), K_NN + 1)
    NB = B * M
    NT = K * NB

    var_per_point = jnp.sum(jnp.var(velocity_in, axis=1, ddof=1), axis=-1)
    mu = jnp.mean(var_per_point, axis=1, keepdims=True)
    sd = jnp.std(var_per_point, axis=1, ddof=1, keepdims=True)
    var_z = (var_per_point - mu) / (sd + 1e-08)
    soft_mask = jax.nn.sigmoid(MASK_SHARPNESS * var_z) * (1.0 - airfoil_mask)
    _, top_idx = jax.lax.top_k(var_per_point, M)

    pos_sel = jnp.take_along_axis(pos, top_idx[:, :, None], axis=1)
    mask_sel = jnp.take_along_axis(soft_mask, top_idx, axis=1)

    pts = lax.stop_gradient(pos_sel)
    knn_local = _sc_knn(pts, B, M).reshape(B, M, K_NN)

    knn_flat = (knn_local + (jnp.arange(B) * M)[:, None, None]).reshape(NB, K_NN)

    pos_cat = pos_sel.reshape(NB, 3)
    rel = pos_cat[knn_flat] - pos_cat[:, None, :]
    dist = jnp.sqrt(jnp.sum(rel * rel, axis=-1, keepdims=True))
    ea = jnp.concatenate([rel, dist], axis=-1).reshape(NB, K_NN * EDGE_DIM)
    ea2 = jnp.concatenate([ea, ea], axis=0)

    v_last = velocity_in[:, -1]
    var_sel = jnp.take_along_axis(var_per_point, top_idx, axis=1)
    ub_sel = jnp.take_along_axis(
        u_base, top_idx[:, None, :, None], axis=2)
    vl_sel = jnp.take_along_axis(v_last, top_idx[:, :, None], axis=1)
    feats = []
    for kk in range(K):
        f = jnp.concatenate(
            [ub_sel[:, kk], vl_sel, pos_sel, var_sel[:, :, None]], axis=-1)
        feats.append(f.reshape(NB, NODE_DIM))
    x = jnp.concatenate(feats, axis=0)

    knn2 = jnp.concatenate([knn_flat, knn_flat + NB], axis=0)
    flat_dst = knn2.reshape(-1)
    deg = jnp.zeros((NT,), jnp.float32).at[flat_dst].add(1.0)

    for i, p in enumerate(params['convs']):
        d_in = NODE_DIM if i == 0 else HIDDEN
        wa = p['w1'][:, :d_in]
        wb = p['w1'][:, d_in:2 * d_in]
        wc = p['w1'][:, 2 * d_in:]
        a = x @ (wa - wb).T
        bv = x @ wb.T
        ag = _sc_gather_rows(a, flat_dst).reshape(NT, K_NN * HIDDEN)
        wcb = jnp.zeros((K_NN * EDGE_DIM, K_NN * HIDDEN), jnp.float32)
        for j in range(K_NN):
            wcb = wcb.at[j * EDGE_DIM:(j + 1) * EDGE_DIM,
                         j * HIDDEN:(j + 1) * HIDDEN].set(wc.T)
        g_edges = _edge_stage(ag, bv, ea2, wcb, p['b1'][None, :])
        s = jnp.zeros((NT, HIDDEN), jnp.float32).at[flat_dst].add(
            g_edges.reshape(NT * K_NN, HIDDEN))
        db2 = deg[:, None] * p['b2'][None, :]
        xres = x if i > 0 else jnp.zeros((NT, HIDDEN), jnp.float32)
        x = _node_stage(s, db2, xres, p['w2'].T,
                        p['ln_g'][None, :], p['ln_b'][None, :],
                        with_res=(i > 0))

    h = params['head']
    mask2 = jnp.concatenate([mask_sel.reshape(NB), mask_sel.reshape(NB)])
    mask_pad = jnp.broadcast_to(mask2[:, None], (NT, 8))
    w2t_pad = jnp.zeros((HIDDEN, 8), jnp.float32).at[:, :3].set(h['w2'].T)
    b2_pad = jnp.zeros((1, 8), jnp.float32).at[0, :3].set(h['b2'])
    delta_pad = _head(x, mask_pad, h['w1'].T, h['b1'][None, :], w2t_pad,
                      b2_pad, h['ln_g'][None, :], h['ln_b'][None, :])
    delta = delta_pad[:, :3]

    delta_kbm = delta.reshape(K, B, M, C).transpose(1, 0, 2, 3)
    bi = jnp.arange(B)[:, None, None]
    ki = jnp.arange(K)[None, :, None]
    delta_full = jnp.zeros((B, K, N, C), u_base.dtype)
    delta_full = delta_full.at[bi, ki, top_idx[:, None, :], :].set(delta_kbm)
    return (u_base + delta_full) * (1.0 - airfoil_mask[:, None, :, None])


# speedup vs baseline: 2.6559x; 1.4348x over previous
import functools
import math

import jax
import jax.numpy as jnp
from jax import lax
from jax.experimental import pallas as pl
from jax.experimental.pallas import tpu as pltpu
from jax.experimental.pallas import tpu_sc as plsc

NODE_DIM = 10
HIDDEN = 64
N_LAYERS = 4
K_NN = 16
TOP_FRACTION = 0.4
MASK_SHARPNESS = 5.0
EDGE_DIM = 4

_INV_SQRT2 = 0.7071067811865476


def _gelu_exact(x):
    return 0.5 * x * (1.0 + lax.erf(x * _INV_SQRT2))


def _knn_body(cols_hbm, out_hbm, px, py, pz, buf, *, M, rows_per_w, nc):
    c = lax.axis_index("c")
    s = lax.axis_index("s")
    wid = s * nc + c
    base = wid * rows_per_w
    b = base // M
    lbase = base - b * M
    cbase = b * 3 * M
    pltpu.sync_copy(cols_hbm.at[pl.ds(cbase, M)], px.at[pl.ds(0, M)])
    pltpu.sync_copy(cols_hbm.at[pl.ds(cbase + M, M)], py.at[pl.ds(0, M)])
    pltpu.sync_copy(cols_hbm.at[pl.ds(cbase + 2 * M, M)], pz.at[pl.ds(0, M)])

    nchunks = M // 16
    inf = jnp.float32(jnp.inf)
    lane = lax.iota(jnp.int32, 16)

    unroll = 10

    def row_body(r, _):
        il = lbase + r
        xi = px[pl.ds(il, 16)][0]
        yi = py[pl.ds(il, 16)][0]
        zi = pz[pl.ds(il, 16)][0]

        def group_body(gi, carry):
            bd, bi, bound = carry
            ds_, is_ = [], []
            hit = None
            for t in range(unroll):
                off = (gi * unroll + t) * 16
                xs = px[pl.ds(off, 16)]
                ys = py[pl.ds(off, 16)]
                zs = pz[pl.ds(off, 16)]
                dx = xs - xi
                dy = ys - yi
                dz = zs - zi
                d = dx * dx + dy * dy + dz * dz
                idxs = off + lane
                d = jnp.where(idxs == il, inf, d)
                m = d < bound
                hit = m if hit is None else (hit | m)
                ds_.append(d)
                is_.append(idxs)

            def merge(args):
                bd0, bi0 = args[0], args[1]
                for t in range(unroll):
                    d0, i0 = args[2 + 2 * t], args[3 + 2 * t]
                    cd, cidx = plsc.sort_key_val(d0, i0, descending=True)
                    take = cd < bd0
                    nd = jnp.where(take, cd, bd0)
                    ni = jnp.where(take, cidx, bi0)
                    bd0, bi0 = plsc.sort_key_val(nd, ni)
                nb = jnp.broadcast_to(bd0[15], (16,))
                return bd0, bi0, nb

            def keep(args):
                return args[0], args[1], bound

            cnt = plsc.all_reduce_population_count(hit)
            pred = (cnt > 0) if jnp.ndim(cnt) == 0 else (cnt[0] > 0)
            flat = (bd, bi) + tuple(
                v for t in range(unroll) for v in (ds_[t], is_[t]))
            return lax.cond(pred, merge, keep, flat)

        init = (jnp.full((16,), inf, jnp.float32),
                jnp.zeros((16,), jnp.int32),
                jnp.full((16,), inf, jnp.float32))
        _, bi, _ = lax.fori_loop(0, nchunks // unroll, group_body, init)
        buf[pl.ds(r * 16, 16)] = bi
        return 0

    lax.fori_loop(0, rows_per_w, row_body, 0)
    pltpu.sync_copy(buf, out_hbm.at[pl.ds(base * K_NN, rows_per_w * K_NN)])


def _sc_knn(pos_sel, B, M):
    info = plsc.get_sparse_core_info()
    nc, ns = info.num_cores, info.num_subcores
    nw = nc * ns
    rows_per_w = (B * M) // nw
    cols = pos_sel.transpose(0, 2, 1).reshape(-1)
    mesh = plsc.VectorSubcoreMesh(core_axis_name="c", subcore_axis_name="s")
    knn = pl.kernel(
        functools.partial(_knn_body, M=M, rows_per_w=rows_per_w, nc=nc),
        out_type=jax.ShapeDtypeStruct((B * M * K_NN,), jnp.int32),
        mesh=mesh,
        compiler_params=pltpu.CompilerParams(needs_layout_passes=False),
        scratch_types=[
            pltpu.VMEM((M + 16,), jnp.float32),
            pltpu.VMEM((M + 16,), jnp.float32),
            pltpu.VMEM((M + 16,), jnp.float32),
            pltpu.VMEM((rows_per_w * K_NN,), jnp.int32),
        ],
    )(cols)
    return knn.reshape(B * M, K_NN)


_GCH = 80
_GNBUF = 4


def _gather_body(tab_hbm, idx_hbm, out_hbm, idx_v, r0, r1, r2, r3,
                 s0, s1, s2, s3, *, rows_per_w, nc):
    c = lax.axis_index("c")
    s = lax.axis_index("s")
    wid = s * nc + c
    base = wid * rows_per_w
    pltpu.sync_copy(idx_hbm.at[pl.ds(base, rows_per_w)], idx_v)
    bufs = (r0, r1, r2, r3)
    sems = (s0, s1, s2, s3)
    nch = rows_per_w // _GCH

    def start(off, b):
        src = tab_hbm.at[idx_v.at[pl.ds(off, _GCH)]]
        pltpu.make_async_copy(src, bufs[b], sems[b]).start()

    def drain(off, b):
        src = tab_hbm.at[idx_v.at[pl.ds(off, _GCH)]]
        pltpu.make_async_copy(src, bufs[b], sems[b]).wait()
        pltpu.sync_copy(bufs[b], out_hbm.at[pl.ds(base + off, _GCH)])

    for b in range(_GNBUF):
        start(b * _GCH, b)

    def group(g, _):
        for b in range(_GNBUF):
            ch = g * _GNBUF + b
            off = pl.multiple_of(ch * _GCH, 16)
            drain(off, b)
            start(pl.multiple_of((ch + _GNBUF) * _GCH, 16), b)
        return 0

    lax.fori_loop(0, nch // _GNBUF - 1, group, 0)
    for b in range(_GNBUF):
        drain((nch - _GNBUF + b) * _GCH, b)


def _sc_gather_rows(tab, idx):
    info = plsc.get_sparse_core_info()
    nc, ns = info.num_cores, info.num_subcores
    nw = nc * ns
    ne = idx.shape[0]
    rows_per_w = ne // nw
    mesh = plsc.VectorSubcoreMesh(core_axis_name="c", subcore_axis_name="s")
    return pl.kernel(
        functools.partial(_gather_body, rows_per_w=rows_per_w, nc=nc),
        out_type=jax.ShapeDtypeStruct((ne, HIDDEN), jnp.float32),
        mesh=mesh,
        compiler_params=pltpu.CompilerParams(needs_layout_passes=False,
                                             use_tc_tiling_on_sc=False),
        scratch_types=(
            [pltpu.VMEM((rows_per_w,), jnp.int32)]
            + [pltpu.VMEM((_GCH, HIDDEN), jnp.float32)] * _GNBUF
            + [pltpu.SemaphoreType.DMA] * _GNBUF
        ),
    )(tab, idx)


def _scatadd_body(g_hbm, idx_hbm, zero_hbm, out_hbm, idxv, wbuf, acc,
                  g0, g1, g2, g3, s0, s1, s2, s3, *, nt2, rows_per_t):
    c = lax.axis_index("c")
    s = lax.axis_index("s")
    bufs = (g0, g1, g2, g3)
    sems = (s0, s1, s2, s3)
    nch = rows_per_t // _GCH
    tail = nt2 + 8 - 15 * 512

    @pl.when(s < 15)
    def _():
        pltpu.sync_copy(zero_hbm, acc.at[pl.ds(s * 512, 512)])

    @pl.when(s == 15)
    def _():
        pltpu.sync_copy(zero_hbm.at[pl.ds(0, tail)],
                        acc.at[pl.ds(15 * 512, tail)])

    pltpu.sync_copy(idx_hbm.at[c * 16 + s], idxv)
    plsc.subcore_barrier()

    def start(j, b):
        src = g_hbm.at[pl.ds(s * rows_per_t + pl.multiple_of(j * _GCH, 16),
                             _GCH)]
        pltpu.make_async_copy(src, bufs[b], sems[b]).start()

    def fin(j, b):
        src = g_hbm.at[pl.ds(s * rows_per_t + pl.multiple_of(j * _GCH, 16),
                             _GCH)]
        pltpu.make_async_copy(src, bufs[b], sems[b]).wait()
        pltpu.sync_copy(bufs[b], acc.at[idxv.at[j]], add=True)

    for b in range(_GNBUF):
        start(b, b)

    def group(g, _):
        for b in range(_GNBUF):
            j = g * _GNBUF + b
            fin(j, b)
            start(j + _GNBUF, b)
        return 0

    lax.fori_loop(0, nch // _GNBUF - 1, group, 0)
    for b in range(_GNBUF):
        fin(nch - _GNBUF + b, b)
    plsc.subcore_barrier()

    @pl.when(s < 15)
    def _():
        pltpu.sync_copy(acc.at[pl.ds(s * 512, 512)], wbuf)
        pltpu.sync_copy(wbuf, out_hbm.at[pl.ds(c * nt2 + s * 512, 512)])

    @pl.when(s == 15)
    def _():
        pltpu.sync_copy(acc.at[pl.ds(15 * 512, nt2 - 15 * 512)],
                        wbuf.at[pl.ds(0, nt2 - 15 * 512)])
        pltpu.sync_copy(wbuf.at[pl.ds(0, nt2 - 15 * 512)],
                        out_hbm.at[pl.ds(c * nt2 + 15 * 512, nt2 - 15 * 512)])


def _sc_scatter_add(g, idx32, zeros_tab, nt):
    info = plsc.get_sparse_core_info()
    nc, ns = info.num_cores, info.num_subcores
    ne = g.shape[0]
    rows_per_t = ne // ns
    nt2 = nt // 2
    mesh = plsc.VectorSubcoreMesh(core_axis_name="c", subcore_axis_name="s")
    return pl.kernel(
        functools.partial(_scatadd_body, nt2=nt2, rows_per_t=rows_per_t),
        out_type=jax.ShapeDtypeStruct((nt, HIDDEN), jnp.float32),
        mesh=mesh,
        compiler_params=pltpu.CompilerParams(needs_layout_passes=False,
                                             use_tc_tiling_on_sc=False),
        scratch_types=(
            [pltpu.VMEM((rows_per_t // _GCH, _GCH), jnp.int32),
             pltpu.VMEM((512, HIDDEN), jnp.float32),
             pltpu.VMEM_SHARED((nt2 + 8, HIDDEN), jnp.float32)]
            + [pltpu.VMEM((_GCH, HIDDEN), jnp.float32)] * _GNBUF
            + [pltpu.SemaphoreType.DMA] * _GNBUF
        ),
    )(g, idx32, zeros_tab)


def _edge_body(ag_ref, bv_ref, ea_ref, wcb_ref, b1_ref, o_ref):
    call = jnp.dot(ea_ref[...], wcb_ref[...],
                   preferred_element_type=jnp.float32)
    base = ag_ref[...] + call
    bvb = bv_ref[...] + b1_ref[...]
    for j in range(K_NN):
        sl = slice(j * HIDDEN, (j + 1) * HIDDEN)
        o_ref[:, sl] = _gelu_exact(base[:, sl] + bvb)


def _edge_stage(ag, bv, ea, wcb, b1):
    n = ag.shape[0]
    R = 800
    grid = n // R
    row = lambda i: (i, 0)
    fixed = lambda i: (0, 0)
    return pl.pallas_call(
        _edge_body,
        grid=(grid,),
        in_specs=[
            pl.BlockSpec((R, K_NN * HIDDEN), row),
            pl.BlockSpec((R, HIDDEN), row),
            pl.BlockSpec((R, K_NN * EDGE_DIM), row),
            pl.BlockSpec((K_NN * EDGE_DIM, K_NN * HIDDEN), fixed),
            pl.BlockSpec((1, HIDDEN), fixed),
        ],
        out_specs=pl.BlockSpec((R, K_NN * HIDDEN), row),
        out_shape=jax.ShapeDtypeStruct((n, K_NN * HIDDEN), jnp.float32),
    )(ag, bv, ea, wcb, b1)


def _node_body(s_ref, db2_ref, xres_ref, w2t_ref, g_ref, bb_ref,
               o_ref, *, with_res):
    out = jnp.dot(s_ref[...], w2t_ref[...],
                  preferred_element_type=jnp.float32) + db2_ref[...]
    mu = jnp.mean(out, axis=-1, keepdims=True)
    var = jnp.mean((out - mu) * (out - mu), axis=-1, keepdims=True)
    out = (out - mu) / jnp.sqrt(var + 1e-5) * g_ref[...] + bb_ref[...]
    if with_res:
        out = out + xres_ref[...]
    o_ref[...] = out


def _node_stage(s, db2, xres, w2t, ln_g, ln_b, with_res):
    n = s.shape[0]
    R = 800
    grid = n // R
    row = lambda i: (i, 0)
    fixed = lambda i: (0, 0)
    return pl.pallas_call(
        functools.partial(_node_body, with_res=with_res),
        grid=(grid,),
        in_specs=[
            pl.BlockSpec((R, HIDDEN), row),
            pl.BlockSpec((R, HIDDEN), row),
            pl.BlockSpec((R, HIDDEN), row),
            pl.BlockSpec((HIDDEN, HIDDEN), fixed),
            pl.BlockSpec((1, HIDDEN), fixed),
            pl.BlockSpec((1, HIDDEN), fixed),
        ],
        out_specs=pl.BlockSpec((R, HIDDEN), row),
        out_shape=jax.ShapeDtypeStruct((n, HIDDEN), jnp.float32),
    )(s, db2, xres, w2t, ln_g, ln_b)


def _head_body(x_ref, m_ref, w1t_ref, b1_ref, w2t_ref, b2_ref, g_ref, bb_ref,
               o_ref):
    x = x_ref[...]
    mu = jnp.mean(x, axis=-1, keepdims=True)
    var = jnp.mean((x - mu) * (x - mu), axis=-1, keepdims=True)
    x = (x - mu) / jnp.sqrt(var + 1e-5) * g_ref[...] + bb_ref[...]
    h = _gelu_exact(jnp.dot(x, w1t_ref[...],
                            preferred_element_type=jnp.float32) + b1_ref[...])
    out = jnp.dot(h, w2t_ref[...],
                  preferred_element_type=jnp.float32) + b2_ref[...]
    o_ref[...] = out * m_ref[...]


def _head(x, mask, w1t, b1, w2t_pad, b2_pad, ln_g, ln_b):
    n = x.shape[0]
    R = 800
    grid = n // R
    row = lambda i: (i, 0)
    fixed = lambda i: (0, 0)
    return pl.pallas_call(
        _head_body,
        grid=(grid,),
        in_specs=[
            pl.BlockSpec((R, HIDDEN), row),
            pl.BlockSpec((R, 8), row),
            pl.BlockSpec((HIDDEN, HIDDEN), fixed),
            pl.BlockSpec((1, HIDDEN), fixed),
            pl.BlockSpec((HIDDEN, 8), fixed),
            pl.BlockSpec((1, 8), fixed),
            pl.BlockSpec((1, HIDDEN), fixed),
            pl.BlockSpec((1, HIDDEN), fixed),
        ],
        out_specs=pl.BlockSpec((R, 8), row),
        out_shape=jax.ShapeDtypeStruct((n, 8), jnp.float32),
    )(x, mask, w1t, b1, w2t_pad, b2_pad, ln_g, ln_b)


def kernel(u_base, pos, velocity_in, airfoil_mask, params):
    B, K, N, C = u_base.shape
    M = max(int(TOP_FRACTION * N), K_NN + 1)
    NB = B * M
    NT = K * NB

    var_per_point = jnp.sum(jnp.var(velocity_in, axis=1, ddof=1), axis=-1)
    mu = jnp.mean(var_per_point, axis=1, keepdims=True)
    sd = jnp.std(var_per_point, axis=1, ddof=1, keepdims=True)
    var_z = (var_per_point - mu) / (sd + 1e-08)
    soft_mask = jax.nn.sigmoid(MASK_SHARPNESS * var_z) * (1.0 - airfoil_mask)
    _, top_idx = jax.lax.top_k(var_per_point, M)

    pos_sel = jnp.take_along_axis(pos, top_idx[:, :, None], axis=1)
    mask_sel = jnp.take_along_axis(soft_mask, top_idx, axis=1)

    pts = lax.stop_gradient(pos_sel)
    knn_local = _sc_knn(pts, B, M).reshape(B, M, K_NN)

    knn_flat = (knn_local + (jnp.arange(B) * M)[:, None, None]).reshape(NB, K_NN)

    pos_cat = pos_sel.reshape(NB, 3)
    rel = pos_cat[knn_flat] - pos_cat[:, None, :]
    dist = jnp.sqrt(jnp.sum(rel * rel, axis=-1, keepdims=True))
    ea = jnp.concatenate([rel, dist], axis=-1).reshape(NB, K_NN * EDGE_DIM)
    ea2 = jnp.concatenate([ea, ea], axis=0)

    v_last = velocity_in[:, -1]
    var_sel = jnp.take_along_axis(var_per_point, top_idx, axis=1)
    ub_sel = jnp.take_along_axis(
        u_base, top_idx[:, None, :, None], axis=2)
    vl_sel = jnp.take_along_axis(v_last, top_idx[:, :, None], axis=1)
    feats = []
    for kk in range(K):
        f = jnp.concatenate(
            [ub_sel[:, kk], vl_sel, pos_sel, var_sel[:, :, None]], axis=-1)
        feats.append(f.reshape(NB, NODE_DIM))
    x = jnp.concatenate(feats, axis=0)

    knn2 = jnp.concatenate([knn_flat, knn_flat + NB], axis=0)
    flat_dst = knn2.reshape(-1)
    deg = jnp.zeros((NT,), jnp.float32).at[flat_dst].add(1.0)
    nt2 = NT // 2
    halves = []
    for ci in range(2):
        loc = flat_dst - ci * nt2
        okm = (loc >= 0) & (loc < nt2)
        halves.append(jnp.where(okm, loc, nt2).reshape(16, -1, _GCH))
    idx32 = jnp.concatenate(halves, axis=0)
    zeros_tab = jnp.zeros((512, HIDDEN), jnp.float32)

    for i, p in enumerate(params['convs']):
        d_in = NODE_DIM if i == 0 else HIDDEN
        wa = p['w1'][:, :d_in]
        wb = p['w1'][:, d_in:2 * d_in]
        wc = p['w1'][:, 2 * d_in:]
        a = x @ (wa - wb).T
        bv = x @ wb.T
        ag = _sc_gather_rows(a, flat_dst).reshape(NT, K_NN * HIDDEN)
        wcb = jnp.zeros((K_NN * EDGE_DIM, K_NN * HIDDEN), jnp.float32)
        for j in range(K_NN):
            wcb = wcb.at[j * EDGE_DIM:(j + 1) * EDGE_DIM,
                         j * HIDDEN:(j + 1) * HIDDEN].set(wc.T)
        g_edges = _edge_stage(ag, bv, ea2, wcb, p['b1'][None, :])
        s = _sc_scatter_add(g_edges.reshape(NT * K_NN, HIDDEN),
                            idx32, zeros_tab, NT)
        db2 = deg[:, None] * p['b2'][None, :]
        xres = x if i > 0 else jnp.zeros((NT, HIDDEN), jnp.float32)
        x = _node_stage(s, db2, xres, p['w2'].T,
                        p['ln_g'][None, :], p['ln_b'][None, :],
                        with_res=(i > 0))

    h = params['head']
    mask2 = jnp.concatenate([mask_sel.reshape(NB), mask_sel.reshape(NB)])
    mask_pad = jnp.broadcast_to(mask2[:, None], (NT, 8))
    w2t_pad = jnp.zeros((HIDDEN, 8), jnp.float32).at[:, :3].set(h['w2'].T)
    b2_pad = jnp.zeros((1, 8), jnp.float32).at[0, :3].set(h['b2'])
    delta_pad = _head(x, mask_pad, h['w1'].T, h['b1'][None, :], w2t_pad,
                      b2_pad, h['ln_g'][None, :], h['ln_b'][None, :])
    delta = delta_pad[:, :3]

    delta_kbm = delta.reshape(K, B, M, C).transpose(1, 0, 2, 3)
    bi = jnp.arange(B)[:, None, None]
    ki = jnp.arange(K)[None, :, None]
    delta_full = jnp.zeros((B, K, N, C), u_base.dtype)
    delta_full = delta_full.at[bi, ki, top_idx[:, None, :], :].set(delta_kbm)
    return (u_base + delta_full) * (1.0 - airfoil_mask[:, None, :, None])
